# baseline (speedup 1.0000x reference)
import jax
import jax.numpy as jnp
from jax import lax
from jax.experimental import pallas as pl
from jax.experimental.pallas import tpu as pltpu
from jax.experimental.pallas import tpu_sc as plsc

_N = 1024 * 100000
_NW = 32
_PW = _N // _NW
_C = 12800
_NCH = _PW // _C
_CM = _C // 4
_G = _C // 64


def _sc_body(logits_hbm, mask_hbm, out_hbm,
             lbuf0, lbuf1, mbuf0, mbuf1, isem0, isem1, osem0, osem1):
    wid = lax.axis_index("s") * 2 + lax.axis_index("c")
    wbase = wid * _PW

    lbufs = (lbuf0, lbuf1)
    mbufs = (mbuf0, mbuf1)
    isems = (isem0, isem1)
    osems = (osem0, osem1)

    iota = lax.iota(jnp.int32, 16)
    lanes4 = jnp.right_shift(iota, 2)
    shamt = 31 - 8 * (iota & 3)

    def start_in(g, b):
        off = pl.multiple_of(wbase + g * _C, 512)
        moff = pl.multiple_of((wbase + g * _C) // 4, 128)
        pltpu.async_copy(logits_hbm.at[pl.ds(off, _C)], lbufs[b], isems[b])
        pltpu.async_copy(mask_hbm.at[pl.ds(moff, _CM)], mbufs[b], isems[b])

    def wait_in(b):
        pltpu.make_async_copy(logits_hbm.at[pl.ds(0, _C)], lbufs[b], isems[b]).wait()
        pltpu.make_async_copy(mask_hbm.at[pl.ds(0, _CM)], mbufs[b], isems[b]).wait()

    def start_out(g, b):
        off = pl.multiple_of(wbase + g * _C, 512)
        pltpu.async_copy(lbufs[b], out_hbm.at[pl.ds(off, _C)], osems[b])

    def wait_out(b):
        pltpu.make_async_copy(lbufs[b], out_hbm.at[pl.ds(0, _C)], osems[b]).wait()

    def compute(b):
        lb, mb = lbufs[b], mbufs[b]

        def grp(j, carry):
            o = j * 64
            mw = mb[pl.ds(j * 16, 16)]
            for s in range(4):
                w = mw.at[lanes4 + 4 * s].get(mode="promise_in_bounds")
                keep = jnp.left_shift(w, shamt) < 0
                x = lb[pl.ds(o + 16 * s, 16)]
                lb[pl.ds(o + 16 * s, 16)] = jnp.where(keep, x, 0.0)
            return carry

        lax.fori_loop(0, _G, grp, 0, unroll=2)

    start_in(0, 0)

    def outer(g2, carry):
        for b in range(2):
            g = 2 * g2 + b

            @pl.when(g + 1 < _NCH)
            def _():
                @pl.when(g >= 1)
                def _():
                    wait_out(1 - b)
                start_in(g + 1, 1 - b)

            wait_in(b)
            compute(b)
            start_out(g, b)
        return carry

    lax.fori_loop(0, _NCH // 2, outer, 0)
    wait_out((_NCH - 1) % 2)


def _make_select():
    mesh = plsc.VectorSubcoreMesh(core_axis_name="c", subcore_axis_name="s")
    return pl.kernel(
        _sc_body,
        out_type=jax.ShapeDtypeStruct((_N,), jnp.float32),
        mesh=mesh,
        scratch_types=[
            pltpu.VMEM((_C,), jnp.float32),
            pltpu.VMEM((_C,), jnp.float32),
            pltpu.VMEM((_CM,), jnp.int32),
            pltpu.VMEM((_CM,), jnp.int32),
            pltpu.SemaphoreType.DMA,
            pltpu.SemaphoreType.DMA,
            pltpu.SemaphoreType.DMA,
            pltpu.SemaphoreType.DMA,
        ],
    )


def kernel(logits, mask):
    lf = logits.reshape(-1)
    mf = mask.reshape(-1).view(jnp.int32)
    out = _make_select()(lf, mf)
    return out.reshape(logits.shape)


# speedup vs baseline: 10.4328x; 10.4328x over previous
import jax
import jax.numpy as jnp
from jax import lax
from jax.experimental import pallas as pl
from jax.experimental.pallas import tpu as pltpu
from jax.experimental.pallas import tpu_sc as plsc

_B = 1024
_V = 100000
_CC = 2560
_NK = 38
_VSC = _CC * _NK
_NW = 32
_RW = _B // _NW
_WPC = _CC // 4
_NT = _CC // 16


def _sc_body(logits_hbm, mask_hbm, out_hbm,
             dbuf0, dbuf1, mbuf0, mbuf1,
             dsem0, dsem1, msem0, msem1, osem0, osem1):
    wid = lax.axis_index("s") * 2 + lax.axis_index("c")
    row0 = pl.multiple_of(wid * _RW, 32)

    dbufs = (dbuf0, dbuf1)
    mbufs = (mbuf0, mbuf1)
    dsems = (dsem0, dsem1)
    msems = (msem0, msem1)
    osems = (osem0, osem1)

    def start_mask(k, p):
        c0 = pl.multiple_of(k * _CC, 128)
        for j in range(4):
            src = (mask_hbm
                   .at[pl.ds(row0, 32), pl.ds(c0 + 640 * j, _CC)]
                   .bitcast(jnp.int32)
                   .at[pl.ds(0, 8), pl.ds(0, _WPC)])
            pltpu.async_copy(src, mbufs[p].at[:, pl.ds(640 * j, _WPC)],
                             msems[p])

    def wait_mask(p):
        for j in range(4):
            pltpu.make_async_copy(
                mask_hbm.at[pl.ds(0, 32), pl.ds(0, _CC)]
                .bitcast(jnp.int32).at[pl.ds(0, 8), pl.ds(0, _WPC)],
                mbufs[p].at[:, pl.ds(640 * j, _WPC)], msems[p]).wait()

    def start_data(k, a, p):
        r = pl.multiple_of(row0 + 8 * a, 8)
        c0 = pl.multiple_of(k * _CC, 128)
        pltpu.async_copy(logits_hbm.at[pl.ds(r, 8), pl.ds(c0, _CC)],
                         dbufs[p], dsems[p])

    def wait_data(p):
        pltpu.make_async_copy(logits_hbm.at[pl.ds(0, 8), pl.ds(0, _CC)],
                              dbufs[p], dsems[p]).wait()

    def start_out(k, a, p):
        r = pl.multiple_of(row0 + 8 * a, 8)
        c0 = pl.multiple_of(k * _CC, 128)
        pltpu.async_copy(dbufs[p], out_hbm.at[pl.ds(r, 8), pl.ds(c0, _CC)],
                         osems[p])

    def wait_out(p):
        pltpu.make_async_copy(dbufs[p], out_hbm.at[pl.ds(0, 8), pl.ds(0, _CC)],
                              osems[p]).wait()

    def compute(a, pd, pm):
        lb = dbufs[pd]
        mb = mbufs[pm]

        def win(t, carry):
            ct = t * 16
            for qh in range(2):
                mw = mb[2 * a + qh, pl.ds(ct, 16)]
                for kk in range(4):
                    keep = jnp.left_shift(mw, 31 - 8 * kk) < 0
                    ri = 4 * qh + kk
                    x = lb[ri, pl.ds(ct, 16)]
                    lb[ri, pl.ds(ct, 16)] = jnp.where(keep, x, 0.0)
            return carry

        lax.fori_loop(0, _NT, win, 0, unroll=2)

    def chunk(k, kc):
        @pl.when(k + 1 < _NK)
        def _():
            start_mask(k + 1, kc ^ 1)
        wait_mask(kc)
        for a in range(4):
            pa = a & 1
            if a == 0:
                @pl.when(k >= 1)
                def _():
                    wait_out(1)
                start_data(k, 1, 1)
            elif a < 3:
                wait_out(pa ^ 1)
                start_data(k, a + 1, pa ^ 1)
            else:
                @pl.when(k + 1 < _NK)
                def _():
                    wait_out(pa ^ 1)
                    start_data(k + 1, 0, pa ^ 1)
            wait_data(pa)
            compute(a, pa, kc)
            start_out(k, a, pa)

    start_mask(0, 0)
    start_data(0, 0, 0)

    def pairs(k2, carry):
        k = 2 * k2
        chunk(k, 0)
        chunk(k + 1, 1)
        return carry

    lax.fori_loop(0, _NK // 2, pairs, 0)

    wait_out(0)
    wait_out(1)


def _make_sc():
    mesh = plsc.VectorSubcoreMesh(core_axis_name="c", subcore_axis_name="s")
    return pl.kernel(
        _sc_body,
        out_type=jax.ShapeDtypeStruct((_B, _V), jnp.float32),
        mesh=mesh,
        scratch_types=[
            pltpu.VMEM((8, _CC), jnp.float32),
            pltpu.VMEM((8, _CC), jnp.float32),
            pltpu.VMEM((8, _CC), jnp.int32),
            pltpu.VMEM((8, _CC), jnp.int32),
            pltpu.SemaphoreType.DMA,
            pltpu.SemaphoreType.DMA,
            pltpu.SemaphoreType.DMA,
            pltpu.SemaphoreType.DMA,
            pltpu.SemaphoreType.DMA,
            pltpu.SemaphoreType.DMA,
        ],
    )


def _tc_tail_body(acc_ref, l_ref, m_ref, o_ref):
    del acc_ref
    o_ref[...] = jnp.where(m_ref[...], l_ref[...], 0.0)


def _tc_tail(acc, logits, mask):
    return pl.pallas_call(
        _tc_tail_body,
        grid=(8, 2),
        in_specs=[
            pl.BlockSpec(memory_space=pl.ANY),
            pl.BlockSpec((128, _CC), lambda i, j: (i, _NK + j)),
            pl.BlockSpec((128, _CC), lambda i, j: (i, _NK + j)),
        ],
        out_specs=pl.BlockSpec((128, _CC), lambda i, j: (i, _NK + j)),
        out_shape=jax.ShapeDtypeStruct((_B, _V), jnp.float32),
        input_output_aliases={0: 0},
    )(acc, logits, mask)


def kernel(logits, mask):
    sc_out = _make_sc()(logits, mask.view(jnp.uint8))
    return _tc_tail(sc_out, logits, mask)
